# baseline (speedup 1.0000x reference)
import jax
import jax.numpy as jnp
from jax.experimental import pallas as pl
from jax.experimental.pallas import tpu as pltpu

_N = 8192
_N_NEW = 4096
_NH = 64
_D3 = 3 * _NH
_DIAG_B = 128
_ROW_B = 128

_f32 = jnp.float32


def _diag_body(node_ref, edge_ref, dn_ref, de_ref, dt_ref):
    b = _DIAG_B
    rows = jax.lax.broadcasted_iota(jnp.int32, (b, b), 0)
    cols = jax.lax.broadcasted_iota(jnp.int32, (b, b), 1)
    eye = rows == cols
    d_node = jnp.sum(jnp.where(eye, node_ref[:, :], 0.0), axis=1, keepdims=True)
    dn_ref[:, :] = d_node
    de_ref[:, :] = jnp.sum(jnp.where(eye, edge_ref[:, :], 0.0), axis=1, keepdims=True)
    dt_ref[:, :] = d_node


def _extract_diags(node_adj, edge_adj):
    nblk = _N // _DIAG_B
    tail0 = _N_NEW // _DIAG_B
    return pl.pallas_call(
        _diag_body,
        grid=(nblk,),
        in_specs=[
            pl.BlockSpec((_DIAG_B, _DIAG_B), lambda i: (i, i)),
            pl.BlockSpec((_DIAG_B, _DIAG_B), lambda i: (i, i)),
        ],
        out_specs=[
            pl.BlockSpec((_DIAG_B, 1), lambda i: (i, 0)),
            pl.BlockSpec((_DIAG_B, 1), lambda i: (i, 0)),
            pl.BlockSpec((_DIAG_B, 1), lambda i: (jnp.maximum(i - tail0, 0), 0)),
        ],
        out_shape=[
            jax.ShapeDtypeStruct((_N, 1), _f32),
            jax.ShapeDtypeStruct((_N, 1), _f32),
            jax.ShapeDtypeStruct((_N_NEW, 1), _f32),
        ],
        compiler_params=pltpu.CompilerParams(
            dimension_semantics=("arbitrary",)),
    )(node_adj, edge_adj)


def _mega_body(node_ref, edge_ref, x0_ref, x1_ref, x2_ref, hin_ref, dt_ref,
               dn_ref, de_ref,
               it0w1_ref, it0b1_ref, it0g_ref, it0bt_ref, it0w2_ref, it0b2_ref,
               it1w1_ref, it1b1_ref, it1g_ref, it1bt_ref, it1w2_ref, it1b2_ref,
               it2w1_ref, it2b1_ref, it2g_ref, it2bt_ref, it2w2_ref, it2b2_ref,
               wzt_ref, uzt_ref, wrt_ref, urt_ref, wnt_ref, unt_ref,
               bz_ref, br_ref, bn_ref, wno_ref, weo_ref, bno_ref, beo_ref,
               ho_ref, z0_ref, z1_ref, z2_ref, y_ref, sig_ref, hf_ref):
    i = pl.program_id(0)

    @pl.when(i == 0)
    def _prologue():
        hf_ref[0:_N_NEW, :] = hin_ref[:, :]
        it_params = (
            (it0w1_ref, it0b1_ref, it0g_ref, it0bt_ref, it0w2_ref, it0b2_ref),
            (it1w1_ref, it1b1_ref, it1g_ref, it1bt_ref, it1w2_ref, it1b2_ref),
            (it2w1_ref, it2b1_ref, it2g_ref, it2bt_ref, it2w2_ref, it2b2_ref),
        )
        xs = (x0_ref, x1_ref, x2_ref)
        for t in range(3):
            w1t, b1, gamma, beta, w2t, b2 = it_params[t]
            h1 = jnp.dot(xs[t][:, :], w1t[:, :],
                         preferred_element_type=_f32) + b1[:, :]
            mu = jnp.mean(h1, axis=0, keepdims=True)
            var = jnp.mean((h1 - mu) ** 2, axis=0, keepdims=True)
            hn = (h1 - mu) / jnp.sqrt(var + 1e-5) * gamma[:, :] + beta[:, :]
            hr = jnp.maximum(hn, 0.0)
            h2 = jnp.dot(hr, w2t[:, :], preferred_element_type=_f32) + b2[:, :]
            hf_ref[_N_NEW:_N, _NH * t:_NH * (t + 1)] = dt_ref[:, :] * h2

    @pl.when(i > 0)
    def _band():
        row0 = (i - 1) * _ROW_B
        a = node_ref[:, :] + edge_ref[:, :]
        m = jnp.dot(a, hf_ref[:, :], preferred_element_type=_f32)
        h = hf_ref[pl.ds(row0, _ROW_B), :]
        dn = dn_ref[pl.ds(row0, _ROW_B), :]
        de = de_ref[pl.ds(row0, _ROW_B), :]
        z = jax.nn.sigmoid(
            jnp.dot(m, wzt_ref[:, :], preferred_element_type=_f32)
            + jnp.dot(h, uzt_ref[:, :], preferred_element_type=_f32)
            + bz_ref[:, :])
        r = jax.nn.sigmoid(
            jnp.dot(m, wrt_ref[:, :], preferred_element_type=_f32)
            + jnp.dot(h, urt_ref[:, :], preferred_element_type=_f32)
            + br_ref[:, :])
        n = jnp.tanh(
            jnp.dot(m, wnt_ref[:, :], preferred_element_type=_f32)
            + jnp.dot(r * h, unt_ref[:, :], preferred_element_type=_f32)
            + bn_ref[:, :])
        ho = (1.0 - z) * h + z * n
        yv = (dn * (jnp.dot(ho, wno_ref[:, :], preferred_element_type=_f32)
                    + bno_ref[:, :])
              + de * (jnp.dot(ho, weo_ref[:, :], preferred_element_type=_f32)
                      + beo_ref[:, :]))
        ho_ref[:, :] = ho
        z0_ref[:, :] = z[:, 0:_NH]
        z1_ref[:, :] = z[:, _NH:2 * _NH]
        z2_ref[:, :] = z[:, 2 * _NH:3 * _NH]
        y_ref[:, :] = yv
        sig_ref[:, :] = jax.nn.sigmoid(yv)


def _block_diag_t(mats):
    out = jnp.zeros((_D3, _D3), _f32)
    for i, m in enumerate(mats):
        out = out.at[_NH * i:_NH * (i + 1), _NH * i:_NH * (i + 1)].set(m.T)
    return out


def _mega(x, h_in, node_adj, edge_adj, dn, de, d_tail, params):
    nband = _N // _ROW_B
    x0 = x[:, 0:8]
    x1 = jnp.pad(x[:, 8:10], ((0, 0), (0, 6)))
    x2 = x[:, 10:138]
    itargs = []
    for t in range(3):
        p = params["it"][t]
        w1 = p["W1"]
        if w1.shape[1] == 2:
            w1 = jnp.pad(w1, ((0, 0), (0, 6)))
        itargs.append(w1.T)
        itargs.append(p["b1"].reshape(1, _NH))
        itargs.append(p["gamma"].reshape(1, _NH))
        itargs.append(p["beta"].reshape(1, _NH))
        itargs.append(p["W2"].T)
        itargs.append(p["b2"].reshape(1, _NH))
    gru = params["gru"]
    wargs = []
    for name in ("Wz", "Uz", "Wr", "Ur", "Wn", "Un"):
        wargs.append(_block_diag_t([gru[t][name] for t in range(3)]))
    for name in ("bz", "br", "bn"):
        wargs.append(jnp.concatenate(
            [gru[t][name] for t in range(3)]).reshape(1, _D3))
    wargs.append(params["out_node"]["W"].T)
    wargs.append(params["out_edge"]["W"].T)
    wargs.append(params["out_node"]["b"].reshape(1, 1))
    wargs.append(params["out_edge"]["b"].reshape(1, 1))

    band = pl.BlockSpec((_ROW_B, _N), lambda i: (jnp.maximum(i - 1, 0), 0))
    full_spec = lambda shape: pl.BlockSpec(shape, lambda i: (0, 0))
    in_specs = [
        band,
        band,
        full_spec((_N_NEW, 8)),
        full_spec((_N_NEW, 8)),
        full_spec((_N_NEW, 128)),
        full_spec((_N_NEW, _D3)),
        full_spec((_N_NEW, 1)),
        full_spec((_N, 1)),
        full_spec((_N, 1)),
    ]
    in_specs += [full_spec(a.shape) for a in itargs]
    in_specs += [full_spec((_D3, _D3))] * 6
    in_specs += [full_spec((1, _D3))] * 3
    in_specs += [full_spec((_D3, 1))] * 2
    in_specs += [full_spec((1, 1))] * 2
    oband = lambda w: pl.BlockSpec((_ROW_B, w), lambda i: (jnp.maximum(i - 1, 0), 0))
    out_specs = [
        oband(_D3), oband(_NH), oband(_NH), oband(_NH), oband(1), oband(1),
    ]
    out_shape = [
        jax.ShapeDtypeStruct((_N, _D3), _f32),
        jax.ShapeDtypeStruct((_N, _NH), _f32),
        jax.ShapeDtypeStruct((_N, _NH), _f32),
        jax.ShapeDtypeStruct((_N, _NH), _f32),
        jax.ShapeDtypeStruct((_N, 1), _f32),
        jax.ShapeDtypeStruct((_N, 1), _f32),
    ]
    return pl.pallas_call(
        _mega_body,
        grid=(nband + 1,),
        in_specs=in_specs,
        out_specs=out_specs,
        out_shape=out_shape,
        scratch_shapes=[pltpu.VMEM((_N, _D3), _f32)],
        compiler_params=pltpu.CompilerParams(
            dimension_semantics=("arbitrary",),
            vmem_limit_bytes=63 * 1024 * 1024),
    )(node_adj, edge_adj, x0, x1, x2, h_in, d_tail, dn, de, *itargs, *wargs)


def kernel(x, h_in, node_adj, edge_adj, params):
    dn, de, d_tail = _extract_diags(node_adj, edge_adj)
    ho, z0, z1, z2, y, sig = _mega(x, h_in, node_adj, edge_adj,
                                   dn, de, d_tail, params)
    return sig, y, ho, (z0, z1, z2)


# speedup vs baseline: 1.0247x; 1.0247x over previous
import jax
import jax.numpy as jnp
from jax.experimental import pallas as pl
from jax.experimental.pallas import tpu as pltpu

_N = 8192
_N_NEW = 4096
_NH = 64
_D3 = 3 * _NH
_DIAG_B = 128
_ROW_B = 256

_f32 = jnp.float32


def _diag_body(node_ref, edge_ref, dn_ref, de_ref, dt_ref):
    b = _DIAG_B
    rows = jax.lax.broadcasted_iota(jnp.int32, (b, b), 0)
    cols = jax.lax.broadcasted_iota(jnp.int32, (b, b), 1)
    eye = rows == cols
    d_node = jnp.sum(jnp.where(eye, node_ref[:, :], 0.0), axis=1, keepdims=True)
    dn_ref[:, :] = d_node
    de_ref[:, :] = jnp.sum(jnp.where(eye, edge_ref[:, :], 0.0), axis=1, keepdims=True)
    dt_ref[:, :] = d_node


def _extract_diags(node_adj, edge_adj):
    nblk = _N // _DIAG_B
    tail0 = _N_NEW // _DIAG_B
    return pl.pallas_call(
        _diag_body,
        grid=(nblk,),
        in_specs=[
            pl.BlockSpec((_DIAG_B, _DIAG_B), lambda i: (i, i)),
            pl.BlockSpec((_DIAG_B, _DIAG_B), lambda i: (i, i)),
        ],
        out_specs=[
            pl.BlockSpec((_DIAG_B, 1), lambda i: (i, 0)),
            pl.BlockSpec((_DIAG_B, 1), lambda i: (i, 0)),
            pl.BlockSpec((_DIAG_B, 1), lambda i: (jnp.maximum(i - tail0, 0), 0)),
        ],
        out_shape=[
            jax.ShapeDtypeStruct((_N, 1), _f32),
            jax.ShapeDtypeStruct((_N, 1), _f32),
            jax.ShapeDtypeStruct((_N_NEW, 1), _f32),
        ],
        compiler_params=pltpu.CompilerParams(
            dimension_semantics=("arbitrary",)),
    )(node_adj, edge_adj)


def _mega_body(node_ref, edge_ref, x0_ref, x1_ref, x2_ref, hin_ref, dt_ref,
               dn_ref, de_ref,
               it0w1_ref, it0b1_ref, it0g_ref, it0bt_ref, it0w2_ref, it0b2_ref,
               it1w1_ref, it1b1_ref, it1g_ref, it1bt_ref, it1w2_ref, it1b2_ref,
               it2w1_ref, it2b1_ref, it2g_ref, it2bt_ref, it2w2_ref, it2b2_ref,
               wzt_ref, uzt_ref, wrt_ref, urt_ref, wnt_ref, unt_ref,
               bz_ref, br_ref, bn_ref, wno_ref, weo_ref, bno_ref, beo_ref,
               ho_ref, z0_ref, z1_ref, z2_ref, y_ref, sig_ref, hf_ref):
    i = pl.program_id(0)

    @pl.when(i == 0)
    def _prologue():
        hf_ref[0:_N_NEW, :] = hin_ref[:, :]
        it_params = (
            (it0w1_ref, it0b1_ref, it0g_ref, it0bt_ref, it0w2_ref, it0b2_ref),
            (it1w1_ref, it1b1_ref, it1g_ref, it1bt_ref, it1w2_ref, it1b2_ref),
            (it2w1_ref, it2b1_ref, it2g_ref, it2bt_ref, it2w2_ref, it2b2_ref),
        )
        xs = (x0_ref, x1_ref, x2_ref)
        for t in range(3):
            w1t, b1, gamma, beta, w2t, b2 = it_params[t]
            h1 = jnp.dot(xs[t][:, :], w1t[:, :],
                         preferred_element_type=_f32) + b1[:, :]
            mu = jnp.mean(h1, axis=0, keepdims=True)
            var = jnp.mean((h1 - mu) ** 2, axis=0, keepdims=True)
            hn = (h1 - mu) / jnp.sqrt(var + 1e-5) * gamma[:, :] + beta[:, :]
            hr = jnp.maximum(hn, 0.0)
            h2 = jnp.dot(hr, w2t[:, :], preferred_element_type=_f32) + b2[:, :]
            hf_ref[_N_NEW:_N, _NH * t:_NH * (t + 1)] = dt_ref[:, :] * h2

    @pl.when(i > 0)
    def _band():
        row0 = (i - 1) * _ROW_B
        a = node_ref[:, :] + edge_ref[:, :]
        m = jnp.dot(a, hf_ref[:, :], preferred_element_type=_f32)
        h = hf_ref[pl.ds(row0, _ROW_B), :]
        dn = dn_ref[:, :]
        de = de_ref[:, :]
        z = jax.nn.sigmoid(
            jnp.dot(m, wzt_ref[:, :], preferred_element_type=_f32)
            + jnp.dot(h, uzt_ref[:, :], preferred_element_type=_f32)
            + bz_ref[:, :])
        r = jax.nn.sigmoid(
            jnp.dot(m, wrt_ref[:, :], preferred_element_type=_f32)
            + jnp.dot(h, urt_ref[:, :], preferred_element_type=_f32)
            + br_ref[:, :])
        n = jnp.tanh(
            jnp.dot(m, wnt_ref[:, :], preferred_element_type=_f32)
            + jnp.dot(r * h, unt_ref[:, :], preferred_element_type=_f32)
            + bn_ref[:, :])
        ho = (1.0 - z) * h + z * n
        yv = (dn * (jnp.dot(ho, wno_ref[:, :], preferred_element_type=_f32)
                    + bno_ref[:, :])
              + de * (jnp.dot(ho, weo_ref[:, :], preferred_element_type=_f32)
                      + beo_ref[:, :]))
        ho_ref[:, :] = ho
        z0_ref[:, :] = z[:, 0:_NH]
        z1_ref[:, :] = z[:, _NH:2 * _NH]
        z2_ref[:, :] = z[:, 2 * _NH:3 * _NH]
        y_ref[:, :] = yv
        sig_ref[:, :] = jax.nn.sigmoid(yv)


def _block_diag_t(mats):
    out = jnp.zeros((_D3, _D3), _f32)
    for i, m in enumerate(mats):
        out = out.at[_NH * i:_NH * (i + 1), _NH * i:_NH * (i + 1)].set(m.T)
    return out


def _mega(x, h_in, node_adj, edge_adj, dn, de, d_tail, params):
    nband = _N // _ROW_B
    x0 = x[:, 0:8]
    x1 = jnp.pad(x[:, 8:10], ((0, 0), (0, 6)))
    x2 = x[:, 10:138]
    itargs = []
    for t in range(3):
        p = params["it"][t]
        w1 = p["W1"]
        if w1.shape[1] == 2:
            w1 = jnp.pad(w1, ((0, 0), (0, 6)))
        itargs.append(w1.T)
        itargs.append(p["b1"].reshape(1, _NH))
        itargs.append(p["gamma"].reshape(1, _NH))
        itargs.append(p["beta"].reshape(1, _NH))
        itargs.append(p["W2"].T)
        itargs.append(p["b2"].reshape(1, _NH))
    gru = params["gru"]
    wargs = []
    for name in ("Wz", "Uz", "Wr", "Ur", "Wn", "Un"):
        wargs.append(_block_diag_t([gru[t][name] for t in range(3)]))
    for name in ("bz", "br", "bn"):
        wargs.append(jnp.concatenate(
            [gru[t][name] for t in range(3)]).reshape(1, _D3))
    wargs.append(params["out_node"]["W"].T)
    wargs.append(params["out_edge"]["W"].T)
    wargs.append(params["out_node"]["b"].reshape(1, 1))
    wargs.append(params["out_edge"]["b"].reshape(1, 1))

    band = pl.BlockSpec((_ROW_B, _N), lambda i: (jnp.maximum(i - 1, 0), 0))
    full_spec = lambda shape: pl.BlockSpec(shape, lambda i: (0, 0))
    in_specs = [
        band,
        band,
        full_spec((_N_NEW, 8)),
        full_spec((_N_NEW, 8)),
        full_spec((_N_NEW, 128)),
        full_spec((_N_NEW, _D3)),
        full_spec((_N_NEW, 1)),
        pl.BlockSpec((_ROW_B, 1), lambda i: (jnp.maximum(i - 1, 0), 0)),
        pl.BlockSpec((_ROW_B, 1), lambda i: (jnp.maximum(i - 1, 0), 0)),
    ]
    in_specs += [full_spec(a.shape) for a in itargs]
    in_specs += [full_spec((_D3, _D3))] * 6
    in_specs += [full_spec((1, _D3))] * 3
    in_specs += [full_spec((_D3, 1))] * 2
    in_specs += [full_spec((1, 1))] * 2
    oband = lambda w: pl.BlockSpec((_ROW_B, w), lambda i: (jnp.maximum(i - 1, 0), 0))
    out_specs = [
        oband(_D3), oband(_NH), oband(_NH), oband(_NH), oband(1), oband(1),
    ]
    out_shape = [
        jax.ShapeDtypeStruct((_N, _D3), _f32),
        jax.ShapeDtypeStruct((_N, _NH), _f32),
        jax.ShapeDtypeStruct((_N, _NH), _f32),
        jax.ShapeDtypeStruct((_N, _NH), _f32),
        jax.ShapeDtypeStruct((_N, 1), _f32),
        jax.ShapeDtypeStruct((_N, 1), _f32),
    ]
    return pl.pallas_call(
        _mega_body,
        grid=(nband + 1,),
        in_specs=in_specs,
        out_specs=out_specs,
        out_shape=out_shape,
        scratch_shapes=[pltpu.VMEM((_N, _D3), _f32)],
        compiler_params=pltpu.CompilerParams(
            dimension_semantics=("arbitrary",),
            vmem_limit_bytes=67000000),
    )(node_adj, edge_adj, x0, x1, x2, h_in, d_tail, dn, de, *itargs, *wargs)


def kernel(x, h_in, node_adj, edge_adj, params):
    dn, de, d_tail = _extract_diags(node_adj, edge_adj)
    ho, z0, z1, z2, y, sig = _mega(x, h_in, node_adj, edge_adj,
                                   dn, de, d_tail, params)
    return sig, y, ho, (z0, z1, z2)


# speedup vs baseline: 1.1129x; 1.0861x over previous
import jax
import jax.numpy as jnp
from jax.experimental import pallas as pl
from jax.experimental.pallas import tpu as pltpu

_N = 8192
_N_NEW = 4096
_NH = 64
_D3 = 3 * _NH
_DIAG_B = 128
_ROW_B = 256

_f32 = jnp.float32


def _diag_body(node_ref, dt_ref):
    b = _DIAG_B
    rows = jax.lax.broadcasted_iota(jnp.int32, (b, b), 0)
    cols = jax.lax.broadcasted_iota(jnp.int32, (b, b), 1)
    eye = rows == cols
    dt_ref[:, :] = jnp.sum(jnp.where(eye, node_ref[:, :], 0.0),
                           axis=1, keepdims=True)


def _extract_dtail(node_adj):
    nblk = _N_NEW // _DIAG_B
    tail0 = _N_NEW // _DIAG_B
    return pl.pallas_call(
        _diag_body,
        grid=(nblk,),
        in_specs=[
            pl.BlockSpec((_DIAG_B, _DIAG_B),
                         lambda i: (i + tail0, i + tail0)),
        ],
        out_specs=[
            pl.BlockSpec((_DIAG_B, 1), lambda i: (i, 0)),
        ],
        out_shape=[
            jax.ShapeDtypeStruct((_N_NEW, 1), _f32),
        ],
        compiler_params=pltpu.CompilerParams(
            dimension_semantics=("arbitrary",)),
    )(node_adj)[0]


def _mega_body(node_ref, edge_ref, x0_ref, x1_ref, x2_ref, hin_ref, dt_ref,
               it0w1_ref, it0b1_ref, it0g_ref, it0bt_ref, it0w2_ref, it0b2_ref,
               it1w1_ref, it1b1_ref, it1g_ref, it1bt_ref, it1w2_ref, it1b2_ref,
               it2w1_ref, it2b1_ref, it2g_ref, it2bt_ref, it2w2_ref, it2b2_ref,
               wzt_ref, uzt_ref, wrt_ref, urt_ref, wnt_ref, unt_ref,
               bz_ref, br_ref, bn_ref, wno_ref, weo_ref, bno_ref, beo_ref,
               ho_ref, z0_ref, z1_ref, z2_ref, y_ref, sig_ref, hf_ref):
    i = pl.program_id(0)

    @pl.when(i == 0)
    def _prologue():
        hf_ref[0:_N_NEW, :] = hin_ref[:, :]
        it_params = (
            (it0w1_ref, it0b1_ref, it0g_ref, it0bt_ref, it0w2_ref, it0b2_ref),
            (it1w1_ref, it1b1_ref, it1g_ref, it1bt_ref, it1w2_ref, it1b2_ref),
            (it2w1_ref, it2b1_ref, it2g_ref, it2bt_ref, it2w2_ref, it2b2_ref),
        )
        xs = (x0_ref, x1_ref, x2_ref)
        for t in range(3):
            w1t, b1, gamma, beta, w2t, b2 = it_params[t]
            h1 = jnp.dot(xs[t][:, :], w1t[:, :],
                         preferred_element_type=_f32) + b1[:, :]
            mu = jnp.mean(h1, axis=0, keepdims=True)
            var = jnp.mean((h1 - mu) ** 2, axis=0, keepdims=True)
            hn = (h1 - mu) / jnp.sqrt(var + 1e-5) * gamma[:, :] + beta[:, :]
            hr = jnp.maximum(hn, 0.0)
            h2 = jnp.dot(hr, w2t[:, :], preferred_element_type=_f32) + b2[:, :]
            hf_ref[_N_NEW:_N, _NH * t:_NH * (t + 1)] = dt_ref[:, :] * h2

    @pl.when(i > 0)
    def _band():
        row0 = (i - 1) * _ROW_B
        a = node_ref[:, :] + edge_ref[:, :]
        m = jnp.dot(a, hf_ref[:, :], preferred_element_type=_f32)
        h = hf_ref[pl.ds(row0, _ROW_B), :]
        rows = jax.lax.broadcasted_iota(jnp.int32, (_ROW_B, _ROW_B), 0)
        cols = jax.lax.broadcasted_iota(jnp.int32, (_ROW_B, _ROW_B), 1)
        eye = rows == cols
        dn = jnp.sum(jnp.where(eye, node_ref[:, pl.ds(row0, _ROW_B)], 0.0),
                     axis=1, keepdims=True)
        de = jnp.sum(jnp.where(eye, edge_ref[:, pl.ds(row0, _ROW_B)], 0.0),
                     axis=1, keepdims=True)
        z = jax.nn.sigmoid(
            jnp.dot(m, wzt_ref[:, :], preferred_element_type=_f32)
            + jnp.dot(h, uzt_ref[:, :], preferred_element_type=_f32)
            + bz_ref[:, :])
        r = jax.nn.sigmoid(
            jnp.dot(m, wrt_ref[:, :], preferred_element_type=_f32)
            + jnp.dot(h, urt_ref[:, :], preferred_element_type=_f32)
            + br_ref[:, :])
        n = jnp.tanh(
            jnp.dot(m, wnt_ref[:, :], preferred_element_type=_f32)
            + jnp.dot(r * h, unt_ref[:, :], preferred_element_type=_f32)
            + bn_ref[:, :])
        ho = (1.0 - z) * h + z * n
        yv = (dn * (jnp.dot(ho, wno_ref[:, :], preferred_element_type=_f32)
                    + bno_ref[:, :])
              + de * (jnp.dot(ho, weo_ref[:, :], preferred_element_type=_f32)
                      + beo_ref[:, :]))
        ho_ref[:, :] = ho
        z0_ref[:, :] = z[:, 0:_NH]
        z1_ref[:, :] = z[:, _NH:2 * _NH]
        z2_ref[:, :] = z[:, 2 * _NH:3 * _NH]
        y_ref[:, :] = yv
        sig_ref[:, :] = jax.nn.sigmoid(yv)


def _block_diag_t(mats):
    out = jnp.zeros((_D3, _D3), _f32)
    for i, m in enumerate(mats):
        out = out.at[_NH * i:_NH * (i + 1), _NH * i:_NH * (i + 1)].set(m.T)
    return out


def _mega(x, h_in, node_adj, edge_adj, d_tail, params):
    nband = _N // _ROW_B
    x0 = x[:, 0:8]
    x1 = jnp.pad(x[:, 8:10], ((0, 0), (0, 6)))
    x2 = x[:, 10:138]
    itargs = []
    for t in range(3):
        p = params["it"][t]
        w1 = p["W1"]
        if w1.shape[1] == 2:
            w1 = jnp.pad(w1, ((0, 0), (0, 6)))
        itargs.append(w1.T)
        itargs.append(p["b1"].reshape(1, _NH))
        itargs.append(p["gamma"].reshape(1, _NH))
        itargs.append(p["beta"].reshape(1, _NH))
        itargs.append(p["W2"].T)
        itargs.append(p["b2"].reshape(1, _NH))
    gru = params["gru"]
    wargs = []
    for name in ("Wz", "Uz", "Wr", "Ur", "Wn", "Un"):
        wargs.append(_block_diag_t([gru[t][name] for t in range(3)]))
    for name in ("bz", "br", "bn"):
        wargs.append(jnp.concatenate(
            [gru[t][name] for t in range(3)]).reshape(1, _D3))
    wargs.append(params["out_node"]["W"].T)
    wargs.append(params["out_edge"]["W"].T)
    wargs.append(params["out_node"]["b"].reshape(1, 1))
    wargs.append(params["out_edge"]["b"].reshape(1, 1))

    band = pl.BlockSpec((_ROW_B, _N), lambda i: (jnp.maximum(i - 1, 0), 0))
    full_spec = lambda shape: pl.BlockSpec(shape, lambda i: (0, 0))
    in_specs = [
        band,
        band,
        full_spec((_N_NEW, 8)),
        full_spec((_N_NEW, 8)),
        full_spec((_N_NEW, 128)),
        full_spec((_N_NEW, _D3)),
        full_spec((_N_NEW, 1)),
    ]
    in_specs += [full_spec(a.shape) for a in itargs]
    in_specs += [full_spec((_D3, _D3))] * 6
    in_specs += [full_spec((1, _D3))] * 3
    in_specs += [full_spec((_D3, 1))] * 2
    in_specs += [full_spec((1, 1))] * 2
    oband = lambda w: pl.BlockSpec((_ROW_B, w), lambda i: (jnp.maximum(i - 1, 0), 0))
    out_specs = [
        oband(_D3), oband(_NH), oband(_NH), oband(_NH), oband(1), oband(1),
    ]
    out_shape = [
        jax.ShapeDtypeStruct((_N, _D3), _f32),
        jax.ShapeDtypeStruct((_N, _NH), _f32),
        jax.ShapeDtypeStruct((_N, _NH), _f32),
        jax.ShapeDtypeStruct((_N, _NH), _f32),
        jax.ShapeDtypeStruct((_N, 1), _f32),
        jax.ShapeDtypeStruct((_N, 1), _f32),
    ]
    return pl.pallas_call(
        _mega_body,
        grid=(nband + 1,),
        in_specs=in_specs,
        out_specs=out_specs,
        out_shape=out_shape,
        scratch_shapes=[pltpu.VMEM((_N, _D3), _f32)],
        compiler_params=pltpu.CompilerParams(
            dimension_semantics=("arbitrary",),
            vmem_limit_bytes=67000000),
    )(node_adj, edge_adj, x0, x1, x2, h_in, d_tail, *itargs, *wargs)


def kernel(x, h_in, node_adj, edge_adj, params):
    d_tail = _extract_dtail(node_adj)
    ho, z0, z1, z2, y, sig = _mega(x, h_in, node_adj, edge_adj,
                                   d_tail, params)
    return sig, y, ho, (z0, z1, z2)
